# baseline (speedup 1.0000x reference)
import functools

import jax
import jax.numpy as jnp
from jax import lax
from jax.experimental import pallas as pl
from jax.experimental.pallas import tpu as pltpu
from jax.experimental.pallas import tpu_sc as plsc

N = 10000
E = 320000
F = 128
G = 64

_NC = 2
_NS = 16
_L = 16

_K = 128
_NCH = E // _K
_CPS = -(-_NCH // _NS)
_RPS = 632
_TAIL = N - (_NS - 1) * _RPS


def _sc_segsum(x, src, dst, tvec, zeros):
    mesh = plsc.VectorSubcoreMesh(core_axis_name="c", subcore_axis_name="s")

    @functools.partial(
        pl.kernel,
        mesh=mesh,
        out_type=jax.ShapeDtypeStruct((2 * N, F), jnp.float32),
        scratch_types=(
            [pltpu.VMEM((_K,), jnp.int32) for _ in range(8)]
            + [pltpu.VMEM((_K, F), jnp.float32) for _ in range(3)]
            + [pltpu.VMEM((_L,), jnp.float32)]
            + [pltpu.VMEM_SHARED((N, F), jnp.float32)]
            + [pltpu.SemaphoreType.DMA for _ in range(15)]
        ),
    )
    def body(x_hbm, src_hbm, dst_hbm, tv_hbm, z_hbm, out_hbm, *scr):
        idx = scr[0:4]
        didx = scr[4:8]
        rows = scr[8:11]
        tv_v = scr[11]
        acc_sh = scr[12]
        ssem = scr[13:17]
        dsem = scr[17:21]
        gsem = scr[21:24]
        csem = scr[24:27]
        tsem = scr[27]
        c = lax.axis_index("c")
        s = lax.axis_index("s")

        pltpu.async_copy(tv_hbm, tv_v, tsem).wait()
        tv = tv_v[...]
        is_f = c == 0

        @pl.when(s < _NS - 1)
        def _():
            pltpu.sync_copy(z_hbm.at[pl.ds(s * _RPS, _RPS)],
                            acc_sh.at[pl.ds(s * _RPS, _RPS)])

        @pl.when(s == _NS - 1)
        def _():
            pltpu.sync_copy(z_hbm.at[pl.ds((_NS - 1) * _RPS, _TAIL)],
                            acc_sh.at[pl.ds((_NS - 1) * _RPS, _TAIL)])

        plsc.subcore_barrier()

        lo = jnp.minimum(s * _CPS, _NCH)
        hi = jnp.minimum(lo + _CPS, _NCH)
        nst = hi - lo
        ebase = lo * _K

        def issue_idx(j, b):
            pltpu.async_copy(src_hbm.at[pl.ds(ebase + j * _K, _K)], idx[b],
                             ssem[b])
            pltpu.async_copy(dst_hbm.at[pl.ds((lo + j) * _K, _K)], didx[b],
                             dsem[b])

        def wait_idx(b):
            pltpu.make_async_copy(src_hbm.at[pl.ds(ebase, _K)], idx[b],
                                  ssem[b]).wait()
            pltpu.make_async_copy(dst_hbm.at[pl.ds(ebase, _K)], didx[b],
                                  dsem[b]).wait()

        def issue_gather(b4, b3):
            pltpu.async_copy(x_hbm.at[idx[b4]], rows[b3], gsem[b3])

        def wait_gather(b4, b3):
            pltpu.make_async_copy(x_hbm.at[idx[b4]], rows[b3],
                                  gsem[b3]).wait()

        def transform(b3):
            def rowpair(i, carry):
                for rr in range(2):
                    r = 2 * i + rr
                    for q in range(F // _L):
                        xv = rows[b3][r, pl.ds(q * _L, _L)]
                        fv = jnp.exp(xv * tv)
                        rows[b3][r, pl.ds(q * _L, _L)] = jnp.where(
                            is_f, fv, xv * fv)
                return carry

            lax.fori_loop(0, _K // 2, rowpair, 0)

        def issue_scatter(b4, b3):
            pltpu.async_copy(rows[b3], acc_sh.at[didx[b4]], csem[b3],
                             add=True)

        def wait_scatter(b4, b3):
            pltpu.make_async_copy(rows[b3], acc_sh.at[didx[b4]],
                                  csem[b3]).wait()

        @pl.when(nst > 0)
        def _():
            issue_idx(0, 0)

        @pl.when(nst > 1)
        def _():
            issue_idx(1, 1)

        @pl.when(nst > 0)
        def _():
            wait_idx(0)
            issue_gather(0, 0)

        def stage(j, u):
            b4, b4n, b4p = u % 4, (u + 1) % 4, (u + 2) % 4
            b3, b3n = u % 3, (u + 1) % 3
            b4s, b3s = (u + 2) % 4, (u + 1) % 3

            @pl.when(j >= 2)
            def _():
                wait_scatter(b4s, b3s)

            @pl.when(j + 2 < nst)
            def _():
                issue_idx(j + 2, b4p)

            @pl.when(j + 1 < nst)
            def _():
                wait_idx(b4n)
                issue_gather(b4n, b3n)

            wait_gather(b4, b3)
            transform(b3)
            issue_scatter(b4, b3)

        def duo(p, carry):
            j0 = 12 * p
            for u in range(12):
                @pl.when(j0 + u < nst)
                def _(u=u):
                    stage(j0 + u, u)
            return carry

        lax.fori_loop(0, (nst + 11) // 12, duo, 0)

        for m4 in range(4):
            for m3 in range(3):
                @pl.when(jnp.logical_and(
                    nst >= 2,
                    jnp.logical_and((nst - 2) % 4 == m4,
                                    (nst - 2) % 3 == m3)))
                def _(m4=m4, m3=m3):
                    wait_scatter(m4, m3)

                @pl.when(jnp.logical_and(
                    nst >= 1,
                    jnp.logical_and((nst - 1) % 4 == m4,
                                    (nst - 1) % 3 == m3)))
                def _(m4=m4, m3=m3):
                    wait_scatter(m4, m3)

        plsc.subcore_barrier()

        @pl.when(s < _NS - 1)
        def _():
            pltpu.sync_copy(acc_sh.at[pl.ds(s * _RPS, _RPS)],
                            out_hbm.at[pl.ds(c * N + s * _RPS, _RPS)])

        @pl.when(s == _NS - 1)
        def _():
            pltpu.sync_copy(acc_sh.at[pl.ds((_NS - 1) * _RPS, _TAIL)],
                            out_hbm.at[pl.ds(c * N + (_NS - 1) * _RPS, _TAIL)])

    return body(x, src, dst, tvec, zeros)


_B = 1000


def _tc_layer(sums, x, W_rel, b_rel, W_root, batch2d, t2d, mlp=None):
    nb = N // _B

    def common(i, den_ref, num_ref, x_ref, wrel_ref, brel_ref, wroot_ref,
               b2_ref, h_ref):
        agg = num_ref[...] / (den_ref[...] + 1e-16)
        y = (
            lax.dot_general(agg, wrel_ref[...], (((1,), (1,)), ((), ())),
                            preferred_element_type=jnp.float32)
            + lax.dot_general(x_ref[...], wroot_ref[...], (((1,), (1,)), ((), ())),
                              preferred_element_type=jnp.float32)
            + brel_ref[...]
        )
        m = jnp.mean(y, axis=1, keepdims=True)
        v = jnp.mean((y - m) ** 2, axis=1, keepdims=True)
        xn = jnp.maximum((y - m) * lax.rsqrt(v + 1e-5), 0.0)

        @pl.when(i == 0)
        def _():
            h_ref[...] = jnp.full((G, F), -jnp.inf, jnp.float32)

        bb = b2_ref[...]
        g0 = bb[0, 0]
        g1 = bb[_B - 1, 0]

        def gbody(gid, carry):
            contrib = jnp.max(
                jnp.where(bb == gid, xn, -jnp.inf), axis=0, keepdims=True)
            h_ref[pl.ds(gid, 1), :] = jnp.maximum(h_ref[pl.ds(gid, 1), :],
                                                  contrib)
            return carry

        lax.fori_loop(g0, g1 + 1, gbody, 0)
        return xn

    in_specs = [
        pl.BlockSpec((_B, F), lambda i: (i, 0)),
        pl.BlockSpec((_B, F), lambda i: (N // _B + i, 0)),
        pl.BlockSpec((_B, F), lambda i: (i, 0)),
        pl.BlockSpec((F, F), lambda i: (0, 0)),
        pl.BlockSpec((1, F), lambda i: (0, 0)),
        pl.BlockSpec((F, F), lambda i: (0, 0)),
        pl.BlockSpec((_B, 1), lambda i: (i, 0)),
        pl.BlockSpec((1, 1), lambda i: (0, 0)),
    ]

    if mlp is None:
        def body(den_ref, num_ref, x_ref, wrel_ref, brel_ref, wroot_ref,
                 b2_ref, t_ref, xn_ref, h_ref):
            i = pl.program_id(0)
            xn = common(i, den_ref, num_ref, x_ref, wrel_ref, brel_ref,
                        wroot_ref, b2_ref, h_ref)
            xn_ref[...] = xn

        return pl.pallas_call(
            body,
            grid=(nb,),
            in_specs=in_specs,
            out_specs=[
                pl.BlockSpec((_B, F), lambda i: (i, 0)),
                pl.BlockSpec((G, F), lambda i: (0, 0)),
            ],
            out_shape=[
                jax.ShapeDtypeStruct((N, F), jnp.float32),
                jax.ShapeDtypeStruct((G, F), jnp.float32),
            ],
        )(sums, sums, x, W_rel, b_rel, W_root, batch2d, t2d)

    h1, h2, w1, b1, w2, b2l = mlp

    def fbody(den_ref, num_ref, x_ref, wrel_ref, brel_ref, wroot_ref,
              b2_ref, t_ref, h1_ref, h2_ref, w1_ref, b1_ref, w2_ref,
              b2l_ref, h_ref, o_ref):
        i = pl.program_id(0)
        common(i, den_ref, num_ref, x_ref, wrel_ref, brel_ref, wroot_ref,
               b2_ref, h_ref)

        @pl.when(i == nb - 1)
        def _():
            h = jnp.concatenate(
                [h1_ref[...], h2_ref[...], h_ref[...]], axis=1)
            a = lax.dot_general(h, w1_ref[...], (((1,), (1,)), ((), ())),
                                preferred_element_type=jnp.float32) + b1_ref[...]
            a = jnp.maximum(a, 0.0)
            o = lax.dot_general(a, w2_ref[...], (((1,), (1,)), ((), ())),
                                preferred_element_type=jnp.float32) + b2l_ref[...]
            n = jnp.sqrt(jnp.sum(o * o, axis=1, keepdims=True))
            o_ref[...] = o / jnp.maximum(n, 1e-12)

    _, o = pl.pallas_call(
        fbody,
        grid=(nb,),
        in_specs=in_specs + [
            pl.BlockSpec((G, F), lambda i: (0, 0)),
            pl.BlockSpec((G, F), lambda i: (0, 0)),
            pl.BlockSpec((2 * F, 3 * F), lambda i: (0, 0)),
            pl.BlockSpec((1, 2 * F), lambda i: (0, 0)),
            pl.BlockSpec((F // 2, 2 * F), lambda i: (0, 0)),
            pl.BlockSpec((1, F // 2), lambda i: (0, 0)),
        ],
        out_specs=[
            pl.BlockSpec((G, F), lambda i: (0, 0)),
            pl.BlockSpec((G, F // 2), lambda i: (0, 0)),
        ],
        out_shape=[
            jax.ShapeDtypeStruct((G, F), jnp.float32),
            jax.ShapeDtypeStruct((G, F // 2), jnp.float32),
        ],
    )(sums, sums, x, W_rel, b_rel, W_root, batch2d, t2d, h1, h2, w1,
      b1.reshape(1, -1), w2, b2l.reshape(1, -1))
    return o


def kernel(x, edge_index, batch, W_rel1, b_rel1, W_root1, W_rel2, b_rel2,
           W_root2, W_rel3, b_rel3, W_root3, t, lin1_w, lin1_b, lin2_w,
           lin2_b):
    src = edge_index[0].astype(jnp.int32)
    dst = edge_index[1].astype(jnp.int32)
    batch2d = batch.astype(jnp.int32).reshape(N, 1)
    t2d = t.reshape(1, 1)
    tvec = jnp.broadcast_to(t, (_L,))

    zeros = jnp.zeros((N, F), jnp.float32)

    sums = _sc_segsum(x, src, dst, tvec, zeros)
    x1, h1 = _tc_layer(sums, x, W_rel1, b_rel1.reshape(1, F), W_root1,
                       batch2d, t2d)

    sums = _sc_segsum(x1, src, dst, tvec, zeros)
    x2, h2 = _tc_layer(sums, x1, W_rel2, b_rel2.reshape(1, F), W_root2,
                       batch2d, t2d)

    sums = _sc_segsum(x2, src, dst, tvec, zeros)
    return _tc_layer(sums, x2, W_rel3, b_rel3.reshape(1, F), W_root3,
                     batch2d, t2d,
                     mlp=(h1, h2, lin1_w, lin1_b, lin2_w, lin2_b))


# speedup vs baseline: 1.2798x; 1.2798x over previous
import functools

import jax
import jax.numpy as jnp
from jax import lax
from jax.experimental import pallas as pl
from jax.experimental.pallas import tpu as pltpu
from jax.experimental.pallas import tpu_sc as plsc

N = 10000
E = 320000
F = 128
G = 64

_NC = 2
_NS = 16
_L = 16

_K = 128
_NCH = E // _K
_CPS = -(-_NCH // _NS)
_RPS = 632
_TAIL = N - (_NS - 1) * _RPS


def _sc_segsum(fg, src2, dst, zeros):
    mesh = plsc.VectorSubcoreMesh(core_axis_name="c", subcore_axis_name="s")

    @functools.partial(
        pl.kernel,
        mesh=mesh,
        out_type=jax.ShapeDtypeStruct((2 * N, F), jnp.float32),
        scratch_types=(
            [pltpu.VMEM((_K,), jnp.int32) for _ in range(8)]
            + [pltpu.VMEM((_K, F), jnp.float32) for _ in range(3)]
            + [pltpu.VMEM_SHARED((N, F), jnp.float32)]
            + [pltpu.SemaphoreType.DMA for _ in range(14)]
        ),
    )
    def body(fg_hbm, src_hbm, dst_hbm, z_hbm, out_hbm, *scr):
        idx = scr[0:4]
        didx = scr[4:8]
        rows = scr[8:11]
        acc_sh = scr[11]
        ssem = scr[12:16]
        dsem = scr[16:20]
        gsem = scr[20:23]
        csem = scr[23:26]
        c = lax.axis_index("c")
        s = lax.axis_index("s")

        @pl.when(s < _NS - 1)
        def _():
            pltpu.sync_copy(z_hbm.at[pl.ds(s * _RPS, _RPS)],
                            acc_sh.at[pl.ds(s * _RPS, _RPS)])

        @pl.when(s == _NS - 1)
        def _():
            pltpu.sync_copy(z_hbm.at[pl.ds((_NS - 1) * _RPS, _TAIL)],
                            acc_sh.at[pl.ds((_NS - 1) * _RPS, _TAIL)])

        plsc.subcore_barrier()

        lo = jnp.minimum(s * _CPS, _NCH)
        hi = jnp.minimum(lo + _CPS, _NCH)
        nst = hi - lo
        ebase = c * E + lo * _K

        def issue_idx(j, b):
            pltpu.async_copy(src_hbm.at[pl.ds(ebase + j * _K, _K)], idx[b],
                             ssem[b])
            pltpu.async_copy(dst_hbm.at[pl.ds((lo + j) * _K, _K)], didx[b],
                             dsem[b])

        def wait_idx(b):
            pltpu.make_async_copy(src_hbm.at[pl.ds(ebase, _K)], idx[b],
                                  ssem[b]).wait()
            pltpu.make_async_copy(dst_hbm.at[pl.ds(ebase, _K)], didx[b],
                                  dsem[b]).wait()

        def issue_gather(b4, b3):
            pltpu.async_copy(fg_hbm.at[idx[b4]], rows[b3], gsem[b3])

        def wait_gather(b4, b3):
            pltpu.make_async_copy(fg_hbm.at[idx[b4]], rows[b3],
                                  gsem[b3]).wait()

        def issue_scatter(b4, b3):
            pltpu.async_copy(rows[b3], acc_sh.at[didx[b4]], csem[b3],
                             add=True)

        def wait_scatter(b4, b3):
            pltpu.make_async_copy(rows[b3], acc_sh.at[didx[b4]],
                                  csem[b3]).wait()

        @pl.when(nst > 0)
        def _():
            issue_idx(0, 0)

        @pl.when(nst > 1)
        def _():
            issue_idx(1, 1)

        @pl.when(nst > 0)
        def _():
            wait_idx(0)
            issue_gather(0, 0)

        def stage(j, u):
            b4, b4n, b4p = u % 4, (u + 1) % 4, (u + 2) % 4
            b3, b3n = u % 3, (u + 1) % 3
            b4s, b3s = (u + 2) % 4, (u + 1) % 3

            @pl.when(j >= 2)
            def _():
                wait_scatter(b4s, b3s)

            @pl.when(j + 2 < nst)
            def _():
                issue_idx(j + 2, b4p)

            @pl.when(j + 1 < nst)
            def _():
                wait_idx(b4n)
                issue_gather(b4n, b3n)

            wait_gather(b4, b3)
            issue_scatter(b4, b3)

        def duo(p, carry):
            j0 = 12 * p
            for u in range(12):
                @pl.when(j0 + u < nst)
                def _(u=u):
                    stage(j0 + u, u)
            return carry

        lax.fori_loop(0, (nst + 11) // 12, duo, 0)

        for m4 in range(4):
            for m3 in range(3):
                @pl.when(jnp.logical_and(
                    nst >= 2,
                    jnp.logical_and((nst - 2) % 4 == m4,
                                    (nst - 2) % 3 == m3)))
                def _(m4=m4, m3=m3):
                    wait_scatter(m4, m3)

                @pl.when(jnp.logical_and(
                    nst >= 1,
                    jnp.logical_and((nst - 1) % 4 == m4,
                                    (nst - 1) % 3 == m3)))
                def _(m4=m4, m3=m3):
                    wait_scatter(m4, m3)

        plsc.subcore_barrier()

        @pl.when(s < _NS - 1)
        def _():
            pltpu.sync_copy(acc_sh.at[pl.ds(s * _RPS, _RPS)],
                            out_hbm.at[pl.ds(c * N + s * _RPS, _RPS)])

        @pl.when(s == _NS - 1)
        def _():
            pltpu.sync_copy(acc_sh.at[pl.ds((_NS - 1) * _RPS, _TAIL)],
                            out_hbm.at[pl.ds(c * N + (_NS - 1) * _RPS, _TAIL)])

    return body(fg, src2, dst, zeros)


_B = 1000


def _tc_pre(x, t2d):
    def body(x_ref, t_ref, fg_ref):
        xv = x_ref[...]
        f = jnp.exp(xv * t_ref[0, 0])
        fg_ref[0] = f
        fg_ref[1] = xv * f

    return pl.pallas_call(
        body,
        grid=(N // _B,),
        in_specs=[
            pl.BlockSpec((_B, F), lambda i: (i, 0)),
            pl.BlockSpec((1, 1), lambda i: (0, 0)),
        ],
        out_specs=pl.BlockSpec((2, _B, F), lambda i: (0, i, 0)),
        out_shape=jax.ShapeDtypeStruct((2, N, F), jnp.float32),
    )(x, t2d)


def _tc_layer(sums, x, W_rel, b_rel, W_root, batch2d, t2d, mlp=None):
    nb = N // _B

    def common(i, den_ref, num_ref, x_ref, wrel_ref, brel_ref, wroot_ref,
               b2_ref, h_ref):
        agg = num_ref[...] / (den_ref[...] + 1e-16)
        y = (
            lax.dot_general(agg, wrel_ref[...], (((1,), (1,)), ((), ())),
                            preferred_element_type=jnp.float32)
            + lax.dot_general(x_ref[...], wroot_ref[...], (((1,), (1,)), ((), ())),
                              preferred_element_type=jnp.float32)
            + brel_ref[...]
        )
        m = jnp.mean(y, axis=1, keepdims=True)
        v = jnp.mean((y - m) ** 2, axis=1, keepdims=True)
        xn = jnp.maximum((y - m) * lax.rsqrt(v + 1e-5), 0.0)

        @pl.when(i == 0)
        def _():
            h_ref[...] = jnp.full((G, F), -jnp.inf, jnp.float32)

        bb = b2_ref[...]
        g0 = bb[0, 0]
        g1 = bb[_B - 1, 0]

        def gbody(gid, carry):
            contrib = jnp.max(
                jnp.where(bb == gid, xn, -jnp.inf), axis=0, keepdims=True)
            h_ref[pl.ds(gid, 1), :] = jnp.maximum(h_ref[pl.ds(gid, 1), :],
                                                  contrib)
            return carry

        lax.fori_loop(g0, g1 + 1, gbody, 0)
        return xn

    in_specs = [
        pl.BlockSpec((_B, F), lambda i: (i, 0)),
        pl.BlockSpec((_B, F), lambda i: (N // _B + i, 0)),
        pl.BlockSpec((_B, F), lambda i: (i, 0)),
        pl.BlockSpec((F, F), lambda i: (0, 0)),
        pl.BlockSpec((1, F), lambda i: (0, 0)),
        pl.BlockSpec((F, F), lambda i: (0, 0)),
        pl.BlockSpec((_B, 1), lambda i: (i, 0)),
        pl.BlockSpec((1, 1), lambda i: (0, 0)),
    ]

    if mlp is None:
        def body(den_ref, num_ref, x_ref, wrel_ref, brel_ref, wroot_ref,
                 b2_ref, t_ref, xn_ref, fg_ref, h_ref):
            i = pl.program_id(0)
            xn = common(i, den_ref, num_ref, x_ref, wrel_ref, brel_ref,
                        wroot_ref, b2_ref, h_ref)
            xn_ref[...] = xn
            tt = t_ref[0, 0]
            f = jnp.exp(xn * tt)
            fg_ref[0] = f
            fg_ref[1] = xn * f

        return pl.pallas_call(
            body,
            grid=(nb,),
            in_specs=in_specs,
            out_specs=[
                pl.BlockSpec((_B, F), lambda i: (i, 0)),
                pl.BlockSpec((2, _B, F), lambda i: (0, i, 0)),
                pl.BlockSpec((G, F), lambda i: (0, 0)),
            ],
            out_shape=[
                jax.ShapeDtypeStruct((N, F), jnp.float32),
                jax.ShapeDtypeStruct((2, N, F), jnp.float32),
                jax.ShapeDtypeStruct((G, F), jnp.float32),
            ],
        )(sums, sums, x, W_rel, b_rel, W_root, batch2d, t2d)

    h1, h2, w1, b1, w2, b2l = mlp

    def fbody(den_ref, num_ref, x_ref, wrel_ref, brel_ref, wroot_ref,
              b2_ref, t_ref, h1_ref, h2_ref, w1_ref, b1_ref, w2_ref,
              b2l_ref, h_ref, o_ref):
        i = pl.program_id(0)
        common(i, den_ref, num_ref, x_ref, wrel_ref, brel_ref, wroot_ref,
               b2_ref, h_ref)

        @pl.when(i == nb - 1)
        def _():
            h = jnp.concatenate(
                [h1_ref[...], h2_ref[...], h_ref[...]], axis=1)
            a = lax.dot_general(h, w1_ref[...], (((1,), (1,)), ((), ())),
                                preferred_element_type=jnp.float32) + b1_ref[...]
            a = jnp.maximum(a, 0.0)
            o = lax.dot_general(a, w2_ref[...], (((1,), (1,)), ((), ())),
                                preferred_element_type=jnp.float32) + b2l_ref[...]
            n = jnp.sqrt(jnp.sum(o * o, axis=1, keepdims=True))
            o_ref[...] = o / jnp.maximum(n, 1e-12)

    _, o = pl.pallas_call(
        fbody,
        grid=(nb,),
        in_specs=in_specs + [
            pl.BlockSpec((G, F), lambda i: (0, 0)),
            pl.BlockSpec((G, F), lambda i: (0, 0)),
            pl.BlockSpec((2 * F, 3 * F), lambda i: (0, 0)),
            pl.BlockSpec((1, 2 * F), lambda i: (0, 0)),
            pl.BlockSpec((F // 2, 2 * F), lambda i: (0, 0)),
            pl.BlockSpec((1, F // 2), lambda i: (0, 0)),
        ],
        out_specs=[
            pl.BlockSpec((G, F), lambda i: (0, 0)),
            pl.BlockSpec((G, F // 2), lambda i: (0, 0)),
        ],
        out_shape=[
            jax.ShapeDtypeStruct((G, F), jnp.float32),
            jax.ShapeDtypeStruct((G, F // 2), jnp.float32),
        ],
    )(sums, sums, x, W_rel, b_rel, W_root, batch2d, t2d, h1, h2, w1,
      b1.reshape(1, -1), w2, b2l.reshape(1, -1))
    return o


def kernel(x, edge_index, batch, W_rel1, b_rel1, W_root1, W_rel2, b_rel2,
           W_root2, W_rel3, b_rel3, W_root3, t, lin1_w, lin1_b, lin2_w,
           lin2_b):
    src = edge_index[0].astype(jnp.int32)
    dst = edge_index[1].astype(jnp.int32)
    src2 = jnp.concatenate([src, src + N])
    batch2d = batch.astype(jnp.int32).reshape(N, 1)
    t2d = t.reshape(1, 1)

    zeros = jnp.zeros((N, F), jnp.float32)

    fg = _tc_pre(x, t2d).reshape(2 * N, F)
    sums = _sc_segsum(fg, src2, dst, zeros)
    x1, fg1, h1 = _tc_layer(sums, x, W_rel1, b_rel1.reshape(1, F), W_root1,
                            batch2d, t2d)

    sums = _sc_segsum(fg1.reshape(2 * N, F), src2, dst, zeros)
    x2, fg2, h2 = _tc_layer(sums, x1, W_rel2, b_rel2.reshape(1, F), W_root2,
                            batch2d, t2d)

    sums = _sc_segsum(fg2.reshape(2 * N, F), src2, dst, zeros)
    return _tc_layer(sums, x2, W_rel3, b_rel3.reshape(1, F), W_root3,
                     batch2d, t2d,
                     mlp=(h1, h2, lin1_w, lin1_b, lin2_w, lin2_b))


# speedup vs baseline: 1.2943x; 1.0113x over previous
import functools

import jax
import jax.numpy as jnp
from jax import lax
from jax.experimental import pallas as pl
from jax.experimental.pallas import tpu as pltpu
from jax.experimental.pallas import tpu_sc as plsc

N = 10000
E = 320000
F = 128
G = 64

_NC = 2
_NS = 16
_L = 16

_K = 128
_NCH = E // _K
_CPS = -(-_NCH // _NS)
_RPS = 632
_TAIL = N - (_NS - 1) * _RPS


def _sc_segsum(fg, src2, dst, zeros):
    mesh = plsc.VectorSubcoreMesh(core_axis_name="c", subcore_axis_name="s")

    @functools.partial(
        pl.kernel,
        mesh=mesh,
        out_type=jax.ShapeDtypeStruct((2 * N, F), jnp.float32),
        scratch_types=(
            [pltpu.VMEM((_K,), jnp.int32) for _ in range(8)]
            + [pltpu.VMEM((_K, F), jnp.float32) for _ in range(3)]
            + [pltpu.VMEM_SHARED((N, F), jnp.float32)]
            + [pltpu.SemaphoreType.DMA for _ in range(15)]
        ),
    )
    def body(fg_hbm, src_hbm, dst_hbm, z_hbm, out_hbm, *scr):
        idx = scr[0:4]
        didx = scr[4:8]
        rows = scr[8:11]
        acc_sh = scr[11]
        ssem = scr[12:16]
        dsem = scr[16:20]
        gsem = scr[20:23]
        csem = scr[23:26]
        zsem = scr[26]
        c = lax.axis_index("c")
        s = lax.axis_index("s")

        @pl.when(s < _NS - 1)
        def _():
            pltpu.async_copy(z_hbm.at[pl.ds(s * _RPS, _RPS)],
                             acc_sh.at[pl.ds(s * _RPS, _RPS)], zsem)

        @pl.when(s == _NS - 1)
        def _():
            pltpu.async_copy(z_hbm.at[pl.ds((_NS - 1) * _RPS, _TAIL)],
                             acc_sh.at[pl.ds((_NS - 1) * _RPS, _TAIL)], zsem)

        lo = jnp.minimum(s * _CPS, _NCH)
        hi = jnp.minimum(lo + _CPS, _NCH)
        nst = hi - lo
        ebase = c * E + lo * _K

        def issue_idx(j, b):
            pltpu.async_copy(src_hbm.at[pl.ds(ebase + j * _K, _K)], idx[b],
                             ssem[b])
            pltpu.async_copy(dst_hbm.at[pl.ds((lo + j) * _K, _K)], didx[b],
                             dsem[b])

        def wait_idx(b):
            pltpu.make_async_copy(src_hbm.at[pl.ds(ebase, _K)], idx[b],
                                  ssem[b]).wait()
            pltpu.make_async_copy(dst_hbm.at[pl.ds(ebase, _K)], didx[b],
                                  dsem[b]).wait()

        def issue_gather(b4, b3):
            pltpu.async_copy(fg_hbm.at[idx[b4]], rows[b3], gsem[b3])

        def wait_gather(b4, b3):
            pltpu.make_async_copy(fg_hbm.at[idx[b4]], rows[b3],
                                  gsem[b3]).wait()

        def issue_scatter(b4, b3):
            pltpu.async_copy(rows[b3], acc_sh.at[didx[b4]], csem[b3],
                             add=True)

        def wait_scatter(b4, b3):
            pltpu.make_async_copy(rows[b3], acc_sh.at[didx[b4]],
                                  csem[b3]).wait()

        @pl.when(nst > 0)
        def _():
            issue_idx(0, 0)

        @pl.when(nst > 1)
        def _():
            issue_idx(1, 1)

        @pl.when(nst > 0)
        def _():
            wait_idx(0)
            issue_gather(0, 0)

        @pl.when(s < _NS - 1)
        def _():
            pltpu.make_async_copy(z_hbm.at[pl.ds(s * _RPS, _RPS)],
                                  acc_sh.at[pl.ds(s * _RPS, _RPS)],
                                  zsem).wait()

        @pl.when(s == _NS - 1)
        def _():
            pltpu.make_async_copy(
                z_hbm.at[pl.ds((_NS - 1) * _RPS, _TAIL)],
                acc_sh.at[pl.ds((_NS - 1) * _RPS, _TAIL)], zsem).wait()

        plsc.subcore_barrier()

        def stage(j, u):
            b4, b4n, b4p = u % 4, (u + 1) % 4, (u + 2) % 4
            b3, b3n = u % 3, (u + 1) % 3
            b4s, b3s = (u + 2) % 4, (u + 1) % 3

            @pl.when(j >= 2)
            def _():
                wait_scatter(b4s, b3s)

            @pl.when(j + 2 < nst)
            def _():
                issue_idx(j + 2, b4p)

            @pl.when(j + 1 < nst)
            def _():
                wait_idx(b4n)
                issue_gather(b4n, b3n)

            wait_gather(b4, b3)
            issue_scatter(b4, b3)

        def duo(p, carry):
            j0 = 12 * p
            for u in range(12):
                @pl.when(j0 + u < nst)
                def _(u=u):
                    stage(j0 + u, u)
            return carry

        lax.fori_loop(0, (nst + 11) // 12, duo, 0)

        for m4 in range(4):
            for m3 in range(3):
                @pl.when(jnp.logical_and(
                    nst >= 2,
                    jnp.logical_and((nst - 2) % 4 == m4,
                                    (nst - 2) % 3 == m3)))
                def _(m4=m4, m3=m3):
                    wait_scatter(m4, m3)

                @pl.when(jnp.logical_and(
                    nst >= 1,
                    jnp.logical_and((nst - 1) % 4 == m4,
                                    (nst - 1) % 3 == m3)))
                def _(m4=m4, m3=m3):
                    wait_scatter(m4, m3)

        plsc.subcore_barrier()

        @pl.when(s < _NS - 1)
        def _():
            pltpu.sync_copy(acc_sh.at[pl.ds(s * _RPS, _RPS)],
                            out_hbm.at[pl.ds(c * N + s * _RPS, _RPS)])

        @pl.when(s == _NS - 1)
        def _():
            pltpu.sync_copy(acc_sh.at[pl.ds((_NS - 1) * _RPS, _TAIL)],
                            out_hbm.at[pl.ds(c * N + (_NS - 1) * _RPS, _TAIL)])

    return body(fg, src2, dst, zeros)


_B = 1000


def _tc_pre(x, t2d):
    def body(x_ref, t_ref, fg_ref):
        xv = x_ref[...]
        f = jnp.exp(xv * t_ref[0, 0])
        fg_ref[0] = f
        fg_ref[1] = xv * f

    return pl.pallas_call(
        body,
        grid=(N // _B,),
        in_specs=[
            pl.BlockSpec((_B, F), lambda i: (i, 0)),
            pl.BlockSpec((1, 1), lambda i: (0, 0)),
        ],
        out_specs=pl.BlockSpec((2, _B, F), lambda i: (0, i, 0)),
        out_shape=jax.ShapeDtypeStruct((2, N, F), jnp.float32),
    )(x, t2d)


def _tc_layer(sums, x, W_rel, b_rel, W_root, batch2d, t2d, mlp=None):
    nb = N // _B

    def common(i, den_ref, num_ref, x_ref, wrel_ref, brel_ref, wroot_ref,
               b2_ref, h_ref):
        agg = num_ref[...] / (den_ref[...] + 1e-16)
        y = (
            lax.dot_general(agg, wrel_ref[...], (((1,), (1,)), ((), ())),
                            preferred_element_type=jnp.float32)
            + lax.dot_general(x_ref[...], wroot_ref[...], (((1,), (1,)), ((), ())),
                              preferred_element_type=jnp.float32)
            + brel_ref[...]
        )
        m = jnp.mean(y, axis=1, keepdims=True)
        v = jnp.mean((y - m) ** 2, axis=1, keepdims=True)
        xn = jnp.maximum((y - m) * lax.rsqrt(v + 1e-5), 0.0)

        @pl.when(i == 0)
        def _():
            h_ref[...] = jnp.full((G, F), -jnp.inf, jnp.float32)

        bb = b2_ref[...]
        g0 = bb[0, 0]
        g1 = bb[_B - 1, 0]

        def gbody(gid, carry):
            contrib = jnp.max(
                jnp.where(bb == gid, xn, -jnp.inf), axis=0, keepdims=True)
            h_ref[pl.ds(gid, 1), :] = jnp.maximum(h_ref[pl.ds(gid, 1), :],
                                                  contrib)
            return carry

        lax.fori_loop(g0, g1 + 1, gbody, 0)
        return xn

    in_specs = [
        pl.BlockSpec((_B, F), lambda i: (i, 0)),
        pl.BlockSpec((_B, F), lambda i: (N // _B + i, 0)),
        pl.BlockSpec((_B, F), lambda i: (i, 0)),
        pl.BlockSpec((F, F), lambda i: (0, 0)),
        pl.BlockSpec((1, F), lambda i: (0, 0)),
        pl.BlockSpec((F, F), lambda i: (0, 0)),
        pl.BlockSpec((_B, 1), lambda i: (i, 0)),
        pl.BlockSpec((1, 1), lambda i: (0, 0)),
    ]

    if mlp is None:
        def body(den_ref, num_ref, x_ref, wrel_ref, brel_ref, wroot_ref,
                 b2_ref, t_ref, xn_ref, fg_ref, h_ref):
            i = pl.program_id(0)
            xn = common(i, den_ref, num_ref, x_ref, wrel_ref, brel_ref,
                        wroot_ref, b2_ref, h_ref)
            xn_ref[...] = xn
            tt = t_ref[0, 0]
            f = jnp.exp(xn * tt)
            fg_ref[0] = f
            fg_ref[1] = xn * f

        return pl.pallas_call(
            body,
            grid=(nb,),
            in_specs=in_specs,
            out_specs=[
                pl.BlockSpec((_B, F), lambda i: (i, 0)),
                pl.BlockSpec((2, _B, F), lambda i: (0, i, 0)),
                pl.BlockSpec((G, F), lambda i: (0, 0)),
            ],
            out_shape=[
                jax.ShapeDtypeStruct((N, F), jnp.float32),
                jax.ShapeDtypeStruct((2, N, F), jnp.float32),
                jax.ShapeDtypeStruct((G, F), jnp.float32),
            ],
        )(sums, sums, x, W_rel, b_rel, W_root, batch2d, t2d)

    h1, h2, w1, b1, w2, b2l = mlp

    def fbody(den_ref, num_ref, x_ref, wrel_ref, brel_ref, wroot_ref,
              b2_ref, t_ref, h1_ref, h2_ref, w1_ref, b1_ref, w2_ref,
              b2l_ref, h_ref, o_ref):
        i = pl.program_id(0)
        common(i, den_ref, num_ref, x_ref, wrel_ref, brel_ref, wroot_ref,
               b2_ref, h_ref)

        @pl.when(i == nb - 1)
        def _():
            h = jnp.concatenate(
                [h1_ref[...], h2_ref[...], h_ref[...]], axis=1)
            a = lax.dot_general(h, w1_ref[...], (((1,), (1,)), ((), ())),
                                preferred_element_type=jnp.float32) + b1_ref[...]
            a = jnp.maximum(a, 0.0)
            o = lax.dot_general(a, w2_ref[...], (((1,), (1,)), ((), ())),
                                preferred_element_type=jnp.float32) + b2l_ref[...]
            n = jnp.sqrt(jnp.sum(o * o, axis=1, keepdims=True))
            o_ref[...] = o / jnp.maximum(n, 1e-12)

    _, o = pl.pallas_call(
        fbody,
        grid=(nb,),
        in_specs=in_specs + [
            pl.BlockSpec((G, F), lambda i: (0, 0)),
            pl.BlockSpec((G, F), lambda i: (0, 0)),
            pl.BlockSpec((2 * F, 3 * F), lambda i: (0, 0)),
            pl.BlockSpec((1, 2 * F), lambda i: (0, 0)),
            pl.BlockSpec((F // 2, 2 * F), lambda i: (0, 0)),
            pl.BlockSpec((1, F // 2), lambda i: (0, 0)),
        ],
        out_specs=[
            pl.BlockSpec((G, F), lambda i: (0, 0)),
            pl.BlockSpec((G, F // 2), lambda i: (0, 0)),
        ],
        out_shape=[
            jax.ShapeDtypeStruct((G, F), jnp.float32),
            jax.ShapeDtypeStruct((G, F // 2), jnp.float32),
        ],
    )(sums, sums, x, W_rel, b_rel, W_root, batch2d, t2d, h1, h2, w1,
      b1.reshape(1, -1), w2, b2l.reshape(1, -1))
    return o


def kernel(x, edge_index, batch, W_rel1, b_rel1, W_root1, W_rel2, b_rel2,
           W_root2, W_rel3, b_rel3, W_root3, t, lin1_w, lin1_b, lin2_w,
           lin2_b):
    src = edge_index[0].astype(jnp.int32)
    dst = edge_index[1].astype(jnp.int32)
    src2 = jnp.concatenate([src, src + N])
    batch2d = batch.astype(jnp.int32).reshape(N, 1)
    t2d = t.reshape(1, 1)

    zeros = jnp.zeros((N, F), jnp.float32)

    fg = _tc_pre(x, t2d).reshape(2 * N, F)
    sums = _sc_segsum(fg, src2, dst, zeros)
    x1, fg1, h1 = _tc_layer(sums, x, W_rel1, b_rel1.reshape(1, F), W_root1,
                            batch2d, t2d)

    sums = _sc_segsum(fg1.reshape(2 * N, F), src2, dst, zeros)
    x2, fg2, h2 = _tc_layer(sums, x1, W_rel2, b_rel2.reshape(1, F), W_root2,
                            batch2d, t2d)

    sums = _sc_segsum(fg2.reshape(2 * N, F), src2, dst, zeros)
    return _tc_layer(sums, x2, W_rel3, b_rel3.reshape(1, F), W_root3,
                     batch2d, t2d,
                     mlp=(h1, h2, lin1_w, lin1_b, lin2_w, lin2_b))
